# Initial kernel scaffold; baseline (speedup 1.0000x reference)
import jax
import jax.numpy as jnp
from jax.experimental import pallas as pl


def kernel(keys, mem_cache, y_cache, tau, slot_q):
    raise NotImplementedError("write your pallas kernel here")


# speedup vs baseline: 1.8234x; 1.8234x over previous
import functools

import jax
import jax.numpy as jnp
from jax.experimental import pallas as pl
from jax.experimental.pallas import tpu as pltpu

KK = 128
DD = 256
TT = 512
NN = 32768
MM = 8
CC = TT + KK


def _norm_logits_body(slot_ref, keys_ref, logits_ref, keysn_ref):
    slot = slot_ref[...]
    sn = slot / (jnp.linalg.norm(slot, axis=-1, keepdims=True) + 1e-6)
    kb = keys_ref[...]
    kn = kb / (jnp.linalg.norm(kb, axis=-1, keepdims=True) + 1e-6)
    keysn_ref[...] = kn
    logits_ref[...] = jax.lax.dot_general(
        sn, kn, (((1,), (1,)), ((), ())), preferred_element_type=jnp.float32)


def _norm_logits(slot_q, keys, bn=4096):
    grid = (NN // bn,)
    return pl.pallas_call(
        _norm_logits_body,
        grid=grid,
        in_specs=[
            pl.BlockSpec((KK, DD), lambda i: (0, 0)),
            pl.BlockSpec((bn, DD), lambda i: (i, 0)),
        ],
        out_specs=[
            pl.BlockSpec((KK, bn), lambda i: (0, i)),
            pl.BlockSpec((bn, DD), lambda i: (i, 0)),
        ],
        out_shape=[
            jax.ShapeDtypeStruct((KK, NN), jnp.float32),
            jax.ShapeDtypeStruct((NN, DD), jnp.float32),
        ],
    )(slot_q, keys)


def _expkeys_body(qp_ref, keysn_ref, out_ref):
    @pl.when(pl.program_id(0) == 0)
    def _():
        out_ref[...] = jnp.zeros_like(out_ref)

    out_ref[...] += jax.lax.dot_general(
        qp_ref[...], keysn_ref[...], (((1,), (0,)), ((), ())),
        preferred_element_type=jnp.float32)


def _expkeys(q_probs, keys_n, bn=4096):
    grid = (NN // bn,)
    return pl.pallas_call(
        _expkeys_body,
        grid=grid,
        in_specs=[
            pl.BlockSpec((KK, bn), lambda i: (0, i)),
            pl.BlockSpec((bn, DD), lambda i: (i, 0)),
        ],
        out_specs=pl.BlockSpec((KK, DD), lambda i: (0, 0)),
        out_shape=jax.ShapeDtypeStruct((KK, DD), jnp.float32),
    )(q_probs, keys_n)


def _gumbel():
    gkey = jax.random.key(42)
    ks = jnp.arange(KK)
    u = jax.vmap(lambda k: jax.random.uniform(
        jax.random.fold_in(gkey, k), (TT,), dtype=jnp.float32))(ks)
    return -jnp.log(-jnp.log(u + 1e-09) + 1e-09)


def kernel(keys, mem_cache, y_cache, tau, slot_q):
    logits_full, keys_n = _norm_logits(slot_q.astype(jnp.float32),
                                       keys.astype(jnp.float32))
    cv, ci = jax.lax.top_k(logits_full, CC)
    g = _gumbel()
    tau_f = jnp.asarray(tau, jnp.float32)

    def step(selected, inp):
        cv_k, ci_k, g_k = inp
        dead = (ci_k[:, None] == selected[None, :]).any(axis=1)
        surv = ~dead
        rank = jnp.cumsum(surv.astype(jnp.int32)) - 1
        is_top = surv & (rank < TT)
        gn = g_k[jnp.clip(rank, 0, TT - 1)]
        s = jnp.where(is_top, (cv_k + gn) / tau_f, -jnp.inf)
        soft = jax.nn.softmax(s)
        soft = jnp.where(is_top, soft, 0.0)
        hard_pos = jnp.argmax(soft)
        idx_hard = ci_k[hard_pos]
        n_sel = (selected >= 0).sum()
        selected = selected.at[n_sel].set(idx_hard)
        return selected, (idx_hard, soft)

    selected0 = jnp.full((KK,), -1, dtype=ci.dtype)
    _, (idx_hard, soft_all) = jax.lax.scan(step, selected0, (cv, ci, g))

    rows = jnp.broadcast_to(jnp.arange(KK)[:, None], (KK, CC))
    q_probs = jnp.zeros((KK, NN), jnp.float32).at[rows, ci].set(soft_all)
    q_st = jnp.zeros((KK, NN), jnp.float32).at[jnp.arange(KK), idx_hard].set(1.0)
    Mem_sel = mem_cache[idx_hard].astype(jnp.float32)
    exp_keys = _expkeys(q_probs, keys_n)
    return (Mem_sel, idx_hard, logits_full, q_probs, q_st, exp_keys)


# speedup vs baseline: 3.4489x; 1.8914x over previous
import functools

import jax
import jax.numpy as jnp
from jax.experimental import pallas as pl
from jax.experimental.pallas import tpu as pltpu

KK = 128
DD = 256
TT = 512
NN = 32768
MM = 8
CC = TT + KK


def _norm_logits_body(slot_ref, keys_ref, logits_ref, keysn_ref):
    slot = slot_ref[...]
    sn = slot / (jnp.linalg.norm(slot, axis=-1, keepdims=True) + 1e-6)
    kb = keys_ref[...]
    kn = kb / (jnp.linalg.norm(kb, axis=-1, keepdims=True) + 1e-6)
    keysn_ref[...] = kn
    logits_ref[...] = jax.lax.dot_general(
        sn, kn, (((1,), (1,)), ((), ())), preferred_element_type=jnp.float32)


def _norm_logits(slot_q, keys, bn=4096):
    grid = (NN // bn,)
    return pl.pallas_call(
        _norm_logits_body,
        grid=grid,
        in_specs=[
            pl.BlockSpec((KK, DD), lambda i: (0, 0)),
            pl.BlockSpec((bn, DD), lambda i: (i, 0)),
        ],
        out_specs=[
            pl.BlockSpec((KK, bn), lambda i: (0, i)),
            pl.BlockSpec((bn, DD), lambda i: (i, 0)),
        ],
        out_shape=[
            jax.ShapeDtypeStruct((KK, NN), jnp.float32),
            jax.ShapeDtypeStruct((NN, DD), jnp.float32),
        ],
    )(slot_q, keys)


def _expkeys_body(qp_ref, keysn_ref, out_ref):
    @pl.when(pl.program_id(0) == 0)
    def _():
        out_ref[...] = jnp.zeros_like(out_ref)

    out_ref[...] += jax.lax.dot_general(
        qp_ref[...], keysn_ref[...], (((1,), (0,)), ((), ())),
        preferred_element_type=jnp.float32)


def _expkeys(q_probs, keys_n, bn=4096):
    grid = (NN // bn,)
    return pl.pallas_call(
        _expkeys_body,
        grid=grid,
        in_specs=[
            pl.BlockSpec((KK, bn), lambda i: (0, i)),
            pl.BlockSpec((bn, DD), lambda i: (i, 0)),
        ],
        out_specs=pl.BlockSpec((KK, DD), lambda i: (0, 0)),
        out_shape=jax.ShapeDtypeStruct((KK, DD), jnp.float32),
    )(q_probs, keys_n)


NEG = -1e30


def _select_body(tau_ref, cv_ref, ci_ref, g_ref, soft_ref, idx_ref):
    tau = tau_ref[0]
    rj = jax.lax.broadcasted_iota(jnp.int32, (CC, CC), 0)
    rc = jax.lax.broadcasted_iota(jnp.int32, (CC, CC), 1)
    Mcum = (rj <= rc).astype(jnp.float32)
    r512 = jax.lax.broadcasted_iota(jnp.int32, (TT, CC), 0)
    lane_i = jax.lax.broadcasted_iota(jnp.int32, (1, CC), 1)
    lane_k = jax.lax.broadcasted_iota(jnp.int32, (1, KK), 1)
    row128 = jax.lax.broadcasted_iota(jnp.int32, (KK, 1), 0)

    def body(k, carry):
        selcol, idxrow = carry
        cv_k = cv_ref[pl.ds(k, 1), :]
        ci_k = ci_ref[pl.ds(k, 1), :]
        g_k = g_ref[pl.ds(k, 1), :]
        eq = (ci_k == selcol).astype(jnp.float32)
        deadcnt = jnp.sum(eq, axis=0, keepdims=True)
        surv = (deadcnt == 0.0)
        survf = surv.astype(jnp.float32)
        cum = jax.lax.dot_general(survf, Mcum, (((1,), (0,)), ((), ())),
                                  preferred_element_type=jnp.float32)
        rank = cum - 1.0
        rank_i = rank.astype(jnp.int32)
        is_top = surv & (rank < float(TT))
        onehotT = (jnp.broadcast_to(rank_i, (TT, CC)) == r512).astype(jnp.float32)
        gn = jax.lax.dot_general(g_k, onehotT, (((1,), (0,)), ((), ())),
                                 preferred_element_type=jnp.float32)
        s = jnp.where(is_top, (cv_k + gn) / tau, NEG)
        m = jnp.max(s, axis=1, keepdims=True)
        e = jnp.exp(s - m)
        soft = e / jnp.sum(e, axis=1, keepdims=True)
        soft = jnp.where(is_top, soft, 0.0)
        m2 = jnp.max(soft, axis=1, keepdims=True)
        pos = jnp.min(jnp.where(soft == m2, lane_i, CC), axis=1, keepdims=True)
        hardmask = lane_i == pos
        idx_hard = jnp.sum(jnp.where(hardmask, ci_k, 0))
        soft_ref[pl.ds(k, 1), :] = soft
        selcol = jnp.where(row128 == k, idx_hard, selcol)
        idxrow = jnp.where(lane_k == k, idx_hard, idxrow)
        return selcol, idxrow

    selcol0 = jnp.full((KK, 1), -1, jnp.int32)
    idxrow0 = jnp.full((1, KK), -1, jnp.int32)
    _, idxrow = jax.lax.fori_loop(0, KK, body, (selcol0, idxrow0))
    idx_ref[...] = idxrow


def _pallas_select(cv, ci, g, tau):
    tau_a = jnp.full((1,), tau, jnp.float32)
    soft, idx = pl.pallas_call(
        _select_body,
        in_specs=[
            pl.BlockSpec(memory_space=pltpu.SMEM),
            pl.BlockSpec(memory_space=pltpu.VMEM),
            pl.BlockSpec(memory_space=pltpu.VMEM),
            pl.BlockSpec(memory_space=pltpu.VMEM),
        ],
        out_specs=[
            pl.BlockSpec(memory_space=pltpu.VMEM),
            pl.BlockSpec(memory_space=pltpu.VMEM),
        ],
        out_shape=[
            jax.ShapeDtypeStruct((KK, CC), jnp.float32),
            jax.ShapeDtypeStruct((1, KK), jnp.int32),
        ],
    )(tau_a, cv, ci, g)
    return soft, idx[0]


def _gumbel():
    gkey = jax.random.key(42)
    ks = jnp.arange(KK)
    u = jax.vmap(lambda k: jax.random.uniform(
        jax.random.fold_in(gkey, k), (TT,), dtype=jnp.float32))(ks)
    return -jnp.log(-jnp.log(u + 1e-09) + 1e-09)


def kernel(keys, mem_cache, y_cache, tau, slot_q):
    logits_full, keys_n = _norm_logits(slot_q.astype(jnp.float32),
                                       keys.astype(jnp.float32))
    cv, ci = jax.lax.top_k(logits_full, CC)
    g = _gumbel()
    tau_f = jnp.asarray(tau, jnp.float32)
    soft_all, idx_hard = _pallas_select(cv, ci, g, tau_f)

    rows = jnp.broadcast_to(jnp.arange(KK)[:, None], (KK, CC))
    q_probs = jnp.zeros((KK, NN), jnp.float32).at[rows, ci].set(soft_all)
    q_st = jnp.zeros((KK, NN), jnp.float32).at[jnp.arange(KK), idx_hard].set(1.0)
    Mem_sel = mem_cache[idx_hard].astype(jnp.float32)
    exp_keys = _expkeys(q_probs, keys_n)
    return (Mem_sel, idx_hard, logits_full, q_probs, q_st, exp_keys)


# speedup vs baseline: 3.8806x; 1.1252x over previous
import functools

import jax
import jax.numpy as jnp
from jax.experimental import pallas as pl
from jax.experimental.pallas import tpu as pltpu

KK = 128
DD = 256
TT = 512
NN = 32768
MM = 8
CC = TT + KK


def _norm_logits_body(slot_ref, keys_ref, logits_ref):
    logits_ref[...] = jax.lax.dot_general(
        slot_ref[...], keys_ref[...], (((1,), (1,)), ((), ())),
        preferred_element_type=jnp.float32)


def _norm_logits(slot_n, keys_n, bn=4096):
    grid = (NN // bn,)
    return pl.pallas_call(
        _norm_logits_body,
        grid=grid,
        in_specs=[
            pl.BlockSpec((KK, DD), lambda i: (0, 0)),
            pl.BlockSpec((bn, DD), lambda i: (i, 0)),
        ],
        out_specs=pl.BlockSpec((KK, bn), lambda i: (0, i)),
        out_shape=jax.ShapeDtypeStruct((KK, NN), jnp.float32),
    )(slot_n, keys_n)


def _expkeys_body(qp_ref, keysn_ref, out_ref):
    @pl.when(pl.program_id(0) == 0)
    def _():
        out_ref[...] = jnp.zeros_like(out_ref)

    out_ref[...] += jax.lax.dot_general(
        qp_ref[...], keysn_ref[...], (((1,), (0,)), ((), ())),
        preferred_element_type=jnp.float32)


def _expkeys(q_probs, keys_n, bn=4096):
    grid = (NN // bn,)
    return pl.pallas_call(
        _expkeys_body,
        grid=grid,
        in_specs=[
            pl.BlockSpec((KK, bn), lambda i: (0, i)),
            pl.BlockSpec((bn, DD), lambda i: (i, 0)),
        ],
        out_specs=pl.BlockSpec((KK, DD), lambda i: (0, 0)),
        out_shape=jax.ShapeDtypeStruct((KK, DD), jnp.float32),
    )(q_probs, keys_n)


NEG = -1e30


def _select_body(tau_ref, cv_ref, ci_ref, g_ref, soft_ref, idx_ref):
    tau = tau_ref[0]
    rj = jax.lax.broadcasted_iota(jnp.int32, (CC, CC), 0)
    rc = jax.lax.broadcasted_iota(jnp.int32, (CC, CC), 1)
    Mcum = (rj <= rc).astype(jnp.float32)
    r512 = jax.lax.broadcasted_iota(jnp.int32, (TT, CC), 0)
    lane_i = jax.lax.broadcasted_iota(jnp.int32, (1, CC), 1)
    lane_k = jax.lax.broadcasted_iota(jnp.int32, (1, KK), 1)
    row128 = jax.lax.broadcasted_iota(jnp.int32, (KK, 1), 0)

    def body(k, carry):
        selcol, idxrow = carry
        cv_k = cv_ref[pl.ds(k, 1), :]
        ci_k = ci_ref[pl.ds(k, 1), :]
        g_k = g_ref[pl.ds(k, 1), :]
        eq = (ci_k == selcol).astype(jnp.float32)
        deadcnt = jnp.sum(eq, axis=0, keepdims=True)
        surv = (deadcnt == 0.0)
        survf = surv.astype(jnp.float32)
        cum = jax.lax.dot_general(survf, Mcum, (((1,), (0,)), ((), ())),
                                  preferred_element_type=jnp.float32)
        rank = cum - 1.0
        rank_i = rank.astype(jnp.int32)
        is_top = surv & (rank < float(TT))
        onehotT = (jnp.broadcast_to(rank_i, (TT, CC)) == r512).astype(jnp.float32)
        gn = jax.lax.dot_general(g_k, onehotT, (((1,), (0,)), ((), ())),
                                 preferred_element_type=jnp.float32)
        s = jnp.where(is_top, (cv_k + gn) / tau, NEG)
        m = jnp.max(s, axis=1, keepdims=True)
        e = jnp.exp(s - m)
        soft = e / jnp.sum(e, axis=1, keepdims=True)
        soft = jnp.where(is_top, soft, 0.0)
        m2 = jnp.max(soft, axis=1, keepdims=True)
        pos = jnp.min(jnp.where(soft == m2, lane_i, CC), axis=1, keepdims=True)
        hardmask = lane_i == pos
        idx_hard = jnp.sum(jnp.where(hardmask, ci_k, 0))
        soft_ref[pl.ds(k, 1), :] = soft
        selcol = jnp.where(row128 == k, idx_hard, selcol)
        idxrow = jnp.where(lane_k == k, idx_hard, idxrow)
        return selcol, idxrow

    selcol0 = jnp.full((KK, 1), -1, jnp.int32)
    idxrow0 = jnp.full((1, KK), -1, jnp.int32)
    _, idxrow = jax.lax.fori_loop(0, KK, body, (selcol0, idxrow0))
    idx_ref[...] = idxrow


def _pallas_select(cv, ci, g, tau):
    tau_a = jnp.full((1,), tau, jnp.float32)
    soft, idx = pl.pallas_call(
        _select_body,
        in_specs=[
            pl.BlockSpec(memory_space=pltpu.SMEM),
            pl.BlockSpec(memory_space=pltpu.VMEM),
            pl.BlockSpec(memory_space=pltpu.VMEM),
            pl.BlockSpec(memory_space=pltpu.VMEM),
        ],
        out_specs=[
            pl.BlockSpec(memory_space=pltpu.VMEM),
            pl.BlockSpec(memory_space=pltpu.VMEM),
        ],
        out_shape=[
            jax.ShapeDtypeStruct((KK, CC), jnp.float32),
            jax.ShapeDtypeStruct((1, KK), jnp.int32),
        ],
    )(tau_a, cv, ci, g)
    return soft, idx[0]


def _gumbel():
    gkey = jax.random.key(42)
    ks = jnp.arange(KK)
    u = jax.vmap(lambda k: jax.random.uniform(
        jax.random.fold_in(gkey, k), (TT,), dtype=jnp.float32))(ks)
    return -jnp.log(-jnp.log(u + 1e-09) + 1e-09)


def kernel(keys, mem_cache, y_cache, tau, slot_q):
    keys_n = keys.astype(jnp.float32)
    keys_n = keys_n / (jnp.linalg.norm(keys_n, axis=-1, keepdims=True) + 1e-6)
    slot_n = slot_q.astype(jnp.float32)
    slot_n = slot_n / (jnp.linalg.norm(slot_n, axis=-1, keepdims=True) + 1e-6)
    logits_full = _norm_logits(slot_n, keys_n)
    NCH = 16
    ch = NN // NCH
    lg3 = logits_full.reshape(KK, NCH, ch)
    cv1, ci1 = jax.lax.top_k(lg3, CC)
    gi1 = (ci1 + (jnp.arange(NCH, dtype=jnp.int32) * ch)[None, :, None])
    cv2 = cv1.reshape(KK, NCH * CC)
    gi2 = gi1.reshape(KK, NCH * CC)
    cv, pos = jax.lax.top_k(cv2, CC)
    ci = jnp.take_along_axis(gi2, pos, axis=1)
    g = _gumbel()
    tau_f = jnp.asarray(tau, jnp.float32)
    soft_all, idx_hard = _pallas_select(cv, ci, g, tau_f)

    rows = jnp.broadcast_to(jnp.arange(KK)[:, None], (KK, CC))
    q_probs = jnp.zeros((KK, NN), jnp.float32).at[rows, ci].set(soft_all)
    q_st = jnp.zeros((KK, NN), jnp.float32).at[jnp.arange(KK), idx_hard].set(1.0)
    Mem_sel = mem_cache[idx_hard].astype(jnp.float32)
    exp_keys = _expkeys(q_probs, keys_n)
    return (Mem_sel, idx_hard, logits_full, q_probs, q_st, exp_keys)
